# baseline (speedup 1.0000x reference)
import functools

import jax
import jax.numpy as jnp
from jax import lax
from jax.experimental import pallas as pl
from jax.experimental.pallas import tpu as pltpu
from jax.experimental.pallas import tpu_sc as plsc

_NC, _NS = 2, 16
_NW = _NC * _NS
_CH = 64
_RING = 3
_LANES = 16


def _proj_body(x_ref, w0_ref, w1_ref, b_ref, y0_ref, y1_ref):
    xb = x_ref[...]
    dn = (((0,), (1,)), ((), ()))
    y0_ref[...] = lax.dot_general(
        xb, w0_ref[...], dn, preferred_element_type=jnp.float32) + b_ref[...]
    y1_ref[...] = lax.dot_general(
        xb, w1_ref[...], dn, preferred_element_type=jnp.float32)


def _make_sc_kernel(vp, c):
    nchunks = vp // (_NW * _CH)
    assert nchunks % _RING == 0
    mesh = plsc.VectorSubcoreMesh(
        core_axis_name="c", subcore_axis_name="s",
        num_cores=_NC, num_subcores=_NS)

    scratch = [pltpu.VMEM((nchunks, 4, _CH), jnp.int32)]
    scratch += [pltpu.VMEM((_CH, c), jnp.float32)
                for _ in range(4 * _RING)]
    scratch += [pltpu.SemaphoreType.DMA for _ in range(2 * _RING)]

    @functools.partial(
        pl.kernel,
        out_type=jax.ShapeDtypeStruct((vp, c), jnp.float32),
        mesh=mesh,
        scratch_types=scratch,
    )
    def sc_fn(y0_hbm, y1_hbm, idx_hbm, out_hbm, idxall, *bufs_sems):
        g = [list(bufs_sems[4 * s:4 * s + 4]) for s in range(_RING)]
        gsem = bufs_sems[4 * _RING:4 * _RING + _RING]
        wsem = bufs_sems[4 * _RING + _RING:]
        wid = lax.axis_index("s") * _NC + lax.axis_index("c")
        base_chunk = wid * nchunks
        pltpu.sync_copy(idx_hbm.at[pl.ds(base_chunk, nchunks)], idxall)

        def fire(cl, s):
            pltpu.async_copy(y0_hbm.at[idxall.at[cl, 0]], g[s][0], gsem[s])
            pltpu.async_copy(y1_hbm.at[idxall.at[cl, 1]], g[s][1], gsem[s])
            pltpu.async_copy(y1_hbm.at[idxall.at[cl, 2]], g[s][2], gsem[s])
            pltpu.async_copy(y1_hbm.at[idxall.at[cl, 3]], g[s][3], gsem[s])

        def drain(sem, buf):
            pltpu.make_async_copy(y0_hbm.at[pl.ds(0, _CH)], buf, sem).wait()

        fire(0, 0)

        def body(jj, carry):
            for p in range(_RING):
                cl = jj * _RING + p
                s = p
                sn = (p + 1) % _RING

                @pl.when(cl + 1 < nchunks)
                def _():
                    @pl.when(cl >= _RING - 1)
                    def _():
                        drain(wsem[sn], g[sn][0])
                    fire(cl + 1, sn)

                for _ in range(4):
                    drain(gsem[s], g[s][0])

                ba, bb, bc, bd = g[s]

                def row(r, rcarry):
                    for q in range(c // _LANES):
                        sl = pl.ds(q * _LANES, _LANES)
                        ba[r, sl] = ba[r, sl] + bb[r, sl] + bc[r, sl] + bd[r, sl]
                    return rcarry

                lax.fori_loop(0, _CH, row, 0)
                pltpu.async_copy(
                    ba, out_hbm.at[pl.ds((base_chunk + cl) * _CH, _CH)],
                    wsem[s])
            return carry

        lax.fori_loop(0, nchunks // _RING, body, 0)
        for s in range(_RING):
            drain(wsem[s], g[s][0])

    return sc_fn


def kernel(x, Gi, W, b):
    bsz, cin, v, _ = x.shape
    cout = W.shape[0]
    x2d = x[0, :, :, 0]
    w0 = W[:, :, 0, 0]
    w1 = W[:, :, 0, 1]
    b2 = b.reshape(1, cout)

    vt = 2048
    y0, y1 = pl.pallas_call(
        _proj_body,
        grid=(pl.cdiv(v, vt),),
        in_specs=[
            pl.BlockSpec((cin, vt), lambda i: (0, i)),
            pl.BlockSpec((cout, cin), lambda i: (0, 0)),
            pl.BlockSpec((cout, cin), lambda i: (0, 0)),
            pl.BlockSpec((1, cout), lambda i: (0, 0)),
        ],
        out_specs=[
            pl.BlockSpec((vt, cout), lambda i: (i, 0)),
            pl.BlockSpec((vt, cout), lambda i: (i, 0)),
        ],
        out_shape=[
            jax.ShapeDtypeStruct((v, cout), jnp.float32),
            jax.ShapeDtypeStruct((v, cout), jnp.float32),
        ],
        compiler_params=pltpu.CompilerParams(
            dimension_semantics=("arbitrary",)),
    )(x2d, w0, w1, b2)

    grain = _NW * _CH * _RING
    vp = ((v + grain - 1) // grain) * grain
    idx = Gi[0, :, :4].astype(jnp.int32)
    idx = jnp.pad(idx, ((0, vp - v), (0, 0)))
    idxb = idx.T.reshape(4, vp // _CH, _CH).transpose(1, 0, 2)

    out_t = _make_sc_kernel(vp, cout)(y0, y1, idxb)
    out = out_t[:v].T
    return out[None, :, :, None]
